# baseline (speedup 1.0000x reference)
import functools

import jax
import jax.numpy as jnp
from jax import lax
from jax.experimental import pallas as pl
from jax.experimental.pallas import tpu as pltpu
from jax.experimental.pallas import tpu_sc as plsc

NCORES = 2
NSUB = 16
NWORK = NCORES * NSUB
DCHUNK = 128
ACHUNK = 128
LANES = 16
NBLOCKS = 8
EPS = 1e-5


def _sc_mesh():
    return plsc.VectorSubcoreMesh(core_axis_name="c", subcore_axis_name="s")


def _sc_degree(dst2d, ew2d, n_pad):
    k_chunks = dst2d.shape[0] // NWORK
    rows_per_tile = n_pad // NSUB
    fire = 8

    @functools.partial(
        pl.kernel,
        out_type=jax.ShapeDtypeStruct((NCORES, n_pad), jnp.float32),
        mesh=_sc_mesh(),
        scratch_types=[
            pltpu.VMEM((k_chunks, DCHUNK), jnp.int32),
            pltpu.VMEM((k_chunks, DCHUNK), jnp.float32),
            pltpu.VMEM((rows_per_tile,), jnp.float32),
            pltpu.VMEM_SHARED((n_pad,), jnp.float32),
            pltpu.SemaphoreType.DMA,
        ],
    )
    def deg_kernel(dst_hbm, ew_hbm, out_hbm, idx_v, ew_v, z_v, acc, sem):
        cid = lax.axis_index("c")
        sid = lax.axis_index("s")

        def zloop(i, carry):
            z_v[pl.ds(i * LANES, LANES)] = jnp.zeros((LANES,), jnp.float32)
            return carry

        lax.fori_loop(0, rows_per_tile // LANES, zloop, 0)
        pltpu.sync_copy(z_v, acc.at[pl.ds(sid * rows_per_tile, rows_per_tile)])

        base = (cid * NSUB + sid) * k_chunks
        pltpu.sync_copy(dst_hbm.at[pl.ds(base, k_chunks)], idx_v)
        pltpu.sync_copy(ew_hbm.at[pl.ds(base, k_chunks)], ew_v)
        plsc.subcore_barrier()

        def body(g, carry):
            descs = []
            for u in range(fire):
                k = g * fire + u
                descs.append(pltpu.async_copy(
                    ew_v.at[k], acc.at[idx_v.at[k]], sem, add=True))
            for dsc in descs:
                dsc.wait()
            return carry

        lax.fori_loop(0, k_chunks // fire, body, 0)
        plsc.subcore_barrier()
        sl = pl.ds(sid * rows_per_tile, rows_per_tile)
        pltpu.sync_copy(acc.at[sl], out_hbm.at[cid, sl])

    return deg_kernel(dst2d, ew2d)


def _sc_aggregate(src, dst, ew, y, n_pad):
    d = y.shape[1]
    ep = src.shape[0]
    k_chunks = ep // (NWORK * ACHUNK)
    npairs = k_chunks // 2
    rows_per_tile = n_pad // NSUB
    half = ACHUNK // 2
    zrows = 16

    @functools.partial(
        pl.kernel,
        out_type=jax.ShapeDtypeStruct((NCORES, n_pad, d), jnp.float32),
        mesh=_sc_mesh(),
        scratch_types=[
            pltpu.VMEM((2, ACHUNK), jnp.int32),
            pltpu.VMEM((2, ACHUNK), jnp.int32),
            pltpu.VMEM((half,), jnp.int32),
            pltpu.VMEM((half,), jnp.int32),
            pltpu.VMEM((half,), jnp.int32),
            pltpu.VMEM((half,), jnp.int32),
            pltpu.VMEM((2, ACHUNK), jnp.float32),
            pltpu.VMEM((ACHUNK, 128), jnp.float32),
            pltpu.VMEM((ACHUNK, 128), jnp.float32),
            pltpu.VMEM((zrows, 128), jnp.float32),
            pltpu.VMEM_SHARED((n_pad, 128), jnp.float32),
            pltpu.SemaphoreType.DMA,
            pltpu.SemaphoreType.DMA,
            pltpu.SemaphoreType.DMA,
            pltpu.SemaphoreType.DMA,
            pltpu.SemaphoreType.DMA,
            pltpu.SemaphoreType.DMA,
            pltpu.SemaphoreType.DMA,
            pltpu.SemaphoreType.DMA,
            pltpu.SemaphoreType.DMA,
        ],
    )
    def agg_kernel(src_hbm, dst_hbm, ew_hbm, y_hbm, out_hbm,
                   sidx, didx, dp0l, dp0h, dp1l, dp1h, ew_v,
                   rows0, rows1, zbuf, acc,
                   isem0, isem1, gsem0, gsem1, s0l, s0h, s1l, s1h, zsem):
        cid = lax.axis_index("c")
        sid = lax.axis_index("s")
        rowsb = (rows0, rows1)
        dpl = (dp0l, dp1l)
        dph = (dp0h, dp1h)
        isem = (isem0, isem1)
        gsem = (gsem0, gsem1)
        ssl = (s0l, s1l)
        ssh = (s0h, s1h)

        def zrow(i, carry):
            for j in range(128 // LANES):
                zbuf[i, pl.ds(j * LANES, LANES)] = jnp.zeros((LANES,), jnp.float32)
            return carry

        lax.fori_loop(0, zrows, zrow, 0)
        zbase = sid * rows_per_tile
        for i in range(rows_per_tile // zrows):
            pltpu.async_copy(zbuf, acc.at[pl.ds(zbase + i * zrows, zrows)], zsem)

        base = (cid * NSUB + sid) * k_chunks

        def fetch_idx(k, b):
            off = (k + base) * ACHUNK
            pltpu.async_copy(src_hbm.at[pl.ds(off, ACHUNK)], sidx.at[b], isem[b])
            pltpu.async_copy(dst_hbm.at[pl.ds(off, ACHUNK)], didx.at[b], isem[b])
            pltpu.async_copy(ew_hbm.at[pl.ds(off, ACHUNK)], ew_v.at[b], isem[b])

        def wait_idx(b):
            pltpu.make_async_copy(
                src_hbm.at[pl.ds(0, ACHUNK)], sidx.at[b], isem[b]).wait()
            pltpu.make_async_copy(
                dst_hbm.at[pl.ds(0, ACHUNK)], didx.at[b], isem[b]).wait()
            pltpu.make_async_copy(
                ew_hbm.at[pl.ds(0, ACHUNK)], ew_v.at[b], isem[b]).wait()

        def scale_half(b, hf):
            rows = rowsb[b]

            def grp(g, c2):
                wv = ew_v[b, pl.ds(g * LANES, LANES)]
                for l in range(LANES):
                    w = wv[l]
                    r = g * LANES + l
                    for j in range(128 // LANES):
                        sl = pl.ds(j * LANES, LANES)
                        rows[r, sl] = rows[r, sl] * w
                return c2

            lax.fori_loop(hf * (half // LANES), (hf + 1) * (half // LANES),
                          grp, 0)

        def gather(b):
            pltpu.async_copy(y_hbm.at[sidx.at[b]], rowsb[b], gsem[b])

        def wait_gather(b):
            pltpu.make_async_copy(
                y_hbm.at[sidx.at[b]], rowsb[b], gsem[b]).wait()

        def scatter_half(b, hf):
            dp = dpl[b] if hf == 0 else dph[b]
            sem = ssl[b] if hf == 0 else ssh[b]
            for j in range(half // LANES):
                dp[pl.ds(j * LANES, LANES)] = (
                    didx[b, pl.ds(hf * half + j * LANES, LANES)])
            pltpu.async_copy(rowsb[b].at[pl.ds(hf * half, half)],
                             acc.at[dp], sem, add=True)

        def wait_scatter_half(b, hf):
            dp = dpl[b] if hf == 0 else dph[b]
            sem = ssl[b] if hf == 0 else ssh[b]
            pltpu.make_async_copy(rowsb[b].at[pl.ds(hf * half, half)],
                                  acc.at[dp], sem).wait()

        for i in range(rows_per_tile // zrows):
            pltpu.make_async_copy(
                zbuf, acc.at[pl.ds(zbase + i * zrows, zrows)], zsem).wait()
        plsc.subcore_barrier()

        fetch_idx(0, 0)
        wait_idx(0)
        gather(0)
        fetch_idx(1, 1)

        def body(j, carry):
            a = j * 2

            @pl.when(j > 0)
            def _():
                wait_scatter_half(1, 0)
                wait_scatter_half(1, 1)

            wait_idx(1)
            gather(1)
            wait_gather(0)
            scale_half(0, 0)
            scatter_half(0, 0)
            scale_half(0, 1)
            scatter_half(0, 1)

            @pl.when(j < npairs - 1)
            def _():
                fetch_idx(a + 2, 0)

            wait_gather(1)
            scale_half(1, 0)
            scatter_half(1, 0)
            wait_scatter_half(0, 0)
            wait_scatter_half(0, 1)

            @pl.when(j < npairs - 1)
            def _():
                wait_idx(0)
                gather(0)

            scale_half(1, 1)
            scatter_half(1, 1)

            @pl.when(j < npairs - 1)
            def _():
                fetch_idx(a + 3, 1)

            return carry

        lax.fori_loop(0, npairs, body, 0)
        wait_scatter_half(1, 0)
        wait_scatter_half(1, 1)
        plsc.subcore_barrier()
        sl = pl.ds(sid * rows_per_tile, rows_per_tile)
        pltpu.sync_copy(acc.at[sl], out_hbm.at[cid, sl])

    return agg_kernel(src, dst, ew, y)


def _tc_prep(xp, W, parts_t):
    n_pad, d = xp.shape
    blk = n_pad // NBLOCKS

    def body(x_ref, w_ref, p_ref, y_ref, dinv_ref):
        deg = p_ref[:, 0:1] + p_ref[:, 1:2] + 1.0
        dinv = lax.rsqrt(deg)
        xw = jnp.dot(x_ref[...], w_ref[...], preferred_element_type=jnp.float32)
        y_ref[...] = xw * dinv
        dinv_ref[...] = dinv

    return pl.pallas_call(
        body,
        grid=(NBLOCKS,),
        in_specs=[
            pl.BlockSpec((blk, d), lambda i: (i, 0)),
            pl.BlockSpec((d, d), lambda i: (0, 0)),
            pl.BlockSpec((blk, NCORES), lambda i: (i, 0)),
        ],
        out_specs=[
            pl.BlockSpec((blk, d), lambda i: (i, 0)),
            pl.BlockSpec((blk, 1), lambda i: (i, 0)),
        ],
        out_shape=[
            jax.ShapeDtypeStruct((n_pad, d), jnp.float32),
            jax.ShapeDtypeStruct((n_pad, 1), jnp.float32),
        ],
    )(xp, W, parts_t)


def _tc_post(parts, y, dinv2, b_row, n_real):
    n_pad, d = y.shape
    blk = n_pad // NBLOCKS

    def body(p_ref, y_ref, dinv_ref, b_ref, t_ref, s_ref):
        i = pl.program_id(0)
        t = dinv_ref[...] * (p_ref[0] + p_ref[1] + y_ref[...]) + b_ref[...]
        t_ref[...] = t
        rowid = lax.broadcasted_iota(jnp.int32, (blk, 1), 0) + i * blk
        tm = jnp.where(rowid < n_real, t, 0.0)

        @pl.when(i == 0)
        def _():
            s_ref[...] = jnp.zeros_like(s_ref)

        s_ref[0:1, :] += jnp.sum(tm, axis=0, keepdims=True)
        s_ref[1:2, :] += jnp.sum(tm * tm, axis=0, keepdims=True)

    return pl.pallas_call(
        body,
        grid=(NBLOCKS,),
        in_specs=[
            pl.BlockSpec((NCORES, blk, d), lambda i: (0, i, 0)),
            pl.BlockSpec((blk, d), lambda i: (i, 0)),
            pl.BlockSpec((blk, 1), lambda i: (i, 0)),
            pl.BlockSpec((1, d), lambda i: (0, 0)),
        ],
        out_specs=[
            pl.BlockSpec((blk, d), lambda i: (i, 0)),
            pl.BlockSpec((2, d), lambda i: (0, 0)),
        ],
        out_shape=[
            jax.ShapeDtypeStruct((n_pad, d), jnp.float32),
            jax.ShapeDtypeStruct((2, d), jnp.float32),
        ],
    )(parts, y, dinv2, b_row)


def _tc_bn_mm(t, stats, g_row, be_row, W, dinv2, n_real):
    n_pad, d = t.shape
    blk = n_pad // NBLOCKS
    inv_n = 1.0 / n_real

    def body(t_ref, s_ref, g_ref, be_ref, w_ref, dinv_ref, y_ref):
        m = s_ref[0:1, :] * inv_n
        var = s_ref[1:2, :] * inv_n - m * m
        scale = lax.rsqrt(var + EPS) * g_ref[...]
        h = jnp.maximum((t_ref[...] - m) * scale + be_ref[...], 0.0)
        hw = jnp.dot(h, w_ref[...], preferred_element_type=jnp.float32)
        y_ref[...] = hw * dinv_ref[...]

    return pl.pallas_call(
        body,
        grid=(NBLOCKS,),
        in_specs=[
            pl.BlockSpec((blk, d), lambda i: (i, 0)),
            pl.BlockSpec((2, d), lambda i: (0, 0)),
            pl.BlockSpec((1, d), lambda i: (0, 0)),
            pl.BlockSpec((1, d), lambda i: (0, 0)),
            pl.BlockSpec((d, d), lambda i: (0, 0)),
            pl.BlockSpec((blk, 1), lambda i: (i, 0)),
        ],
        out_specs=pl.BlockSpec((blk, d), lambda i: (i, 0)),
        out_shape=jax.ShapeDtypeStruct((n_pad, d), jnp.float32),
    )(t, stats, g_row, be_row, W, dinv2)


def _tc_head(t, stats, g_row, be_row, batch2d, Wp1, bp1_row, Wp2, bp2_row,
             n_real, n_graphs):
    n_pad, d = t.shape
    blk = n_pad // NBLOCKS
    inv_n = 1.0 / n_real
    dm = Wp1.shape[1]

    def body(t_ref, s_ref, g_ref, be_ref, b_ref, wp1_ref, bp1_ref, wp2_ref,
             bp2_ref, out_ref, s_acc, c_acc):
        i = pl.program_id(0)
        m = s_ref[0:1, :] * inv_n
        var = s_ref[1:2, :] * inv_n - m * m
        scale = lax.rsqrt(var + EPS) * g_ref[...]
        h = jnp.maximum((t_ref[...] - m) * scale + be_ref[...], 0.0)
        gids = lax.broadcasted_iota(jnp.int32, (1, n_graphs), 1)
        oh = (b_ref[...] == gids).astype(jnp.float32)
        dn = (((0,), (0,)), ((), ()))

        @pl.when(i == 0)
        def _():
            s_acc[...] = jnp.zeros_like(s_acc)
            c_acc[...] = jnp.zeros_like(c_acc)

        s_acc[...] += lax.dot_general(oh, h, dn,
                                      preferred_element_type=jnp.float32)
        c_acc[...] += lax.dot_general(oh, jnp.ones_like(h), dn,
                                      preferred_element_type=jnp.float32)

        @pl.when(i == NBLOCKS - 1)
        def _():
            pooled = s_acc[...] / jnp.maximum(c_acc[...], 1.0)
            z = jnp.dot(pooled, wp1_ref[...],
                        preferred_element_type=jnp.float32) + bp1_ref[...]
            z = jnp.maximum(z, 0.0)
            out_ref[...] = jnp.dot(z, wp2_ref[...],
                                   preferred_element_type=jnp.float32) + bp2_ref[...]

    return pl.pallas_call(
        body,
        grid=(NBLOCKS,),
        in_specs=[
            pl.BlockSpec((blk, d), lambda i: (i, 0)),
            pl.BlockSpec((2, d), lambda i: (0, 0)),
            pl.BlockSpec((1, d), lambda i: (0, 0)),
            pl.BlockSpec((1, d), lambda i: (0, 0)),
            pl.BlockSpec((blk, 1), lambda i: (i, 0)),
            pl.BlockSpec((d, dm), lambda i: (0, 0)),
            pl.BlockSpec((1, dm), lambda i: (0, 0)),
            pl.BlockSpec((dm, 1), lambda i: (0, 0)),
            pl.BlockSpec((1, 1), lambda i: (0, 0)),
        ],
        out_specs=pl.BlockSpec((n_graphs, 1), lambda i: (0, 0)),
        out_shape=jax.ShapeDtypeStruct((n_graphs, 1), jnp.float32),
        scratch_shapes=[
            pltpu.VMEM((n_graphs, d), jnp.float32),
            pltpu.VMEM((n_graphs, d), jnp.float32),
        ],
    )(t, stats, g_row, be_row, batch2d, Wp1, bp1_row, Wp2, bp2_row)


def kernel(x, edge_index, edge_attr, batch,
           W1, b1, g1, be1, W2, b2, g2, be2, Wp1, bp1, Wp2, bp2):
    n, d = x.shape
    e = edge_index.shape[1]
    n_graphs = 64

    n_pad = ((n + 2048) // 2048) * 2048
    estep = NWORK * DCHUNK * 8
    ep = ((e + estep - 1) // estep) * estep

    npad_rows = n_pad - n
    pad_src = jnp.arange(ep - e, dtype=jnp.int32)
    pad_dst = n + jnp.broadcast_to(
        jnp.arange(npad_rows, dtype=jnp.int32),
        ((ep - e + npad_rows - 1) // npad_rows, npad_rows)).reshape(-1)[:ep - e]
    src = jnp.concatenate([edge_index[0], pad_src])
    dst = jnp.concatenate([edge_index[1], pad_dst])
    ew = jnp.concatenate(
        [edge_attr[:, 0], jnp.zeros((ep - e,), jnp.float32)])

    dst2d = dst.reshape(ep // DCHUNK, DCHUNK)
    ew2d = ew.reshape(ep // DCHUNK, DCHUNK)

    xp = jnp.pad(x, ((0, n_pad - n), (0, 0)))
    batch2d = jnp.pad(batch, (0, n_pad - n),
                      constant_values=n_graphs).reshape(n_pad, 1)

    b1r, g1r, be1r = b1.reshape(1, d), g1.reshape(1, d), be1.reshape(1, d)
    b2r, g2r, be2r = b2.reshape(1, d), g2.reshape(1, d), be2.reshape(1, d)
    bp1r = bp1.reshape(1, -1)
    bp2r = bp2.reshape(1, 1)

    deg_parts = _sc_degree(dst2d, ew2d, n_pad)
    parts_t = deg_parts.T

    y1, dinv2 = _tc_prep(xp, W1, parts_t)
    p1 = _sc_aggregate(src, dst, ew, y1, n_pad)
    t1, s1 = _tc_post(p1, y1, dinv2, b1r, n)
    y2 = _tc_bn_mm(t1, s1, g1r, be1r, W2, dinv2, n)
    p2 = _sc_aggregate(src, dst, ew, y2, n_pad)
    t2, s2 = _tc_post(p2, y2, dinv2, b2r, n)
    out = _tc_head(t2, s2, g2r, be2r, batch2d, Wp1, bp1r, Wp2, bp2r,
                   n, n_graphs)
    return out


# speedup vs baseline: 1.0217x; 1.0217x over previous
import functools

import jax
import jax.numpy as jnp
from jax import lax
from jax.experimental import pallas as pl
from jax.experimental.pallas import tpu as pltpu
from jax.experimental.pallas import tpu_sc as plsc

NCORES = 2
NSUB = 16
NWORK = NCORES * NSUB
DCHUNK = 128
ACHUNK = 128
LANES = 16
NBLOCKS = 4
EPS = 1e-5


def _sc_mesh():
    return plsc.VectorSubcoreMesh(core_axis_name="c", subcore_axis_name="s")


def _sc_degree(dst2d, ew2d, n_pad):
    k_chunks = dst2d.shape[0] // NWORK
    rows_per_tile = n_pad // NSUB
    fire = 8

    @functools.partial(
        pl.kernel,
        out_type=jax.ShapeDtypeStruct((NCORES, n_pad), jnp.float32),
        mesh=_sc_mesh(),
        scratch_types=[
            pltpu.VMEM((k_chunks, DCHUNK), jnp.int32),
            pltpu.VMEM((k_chunks, DCHUNK), jnp.float32),
            pltpu.VMEM((rows_per_tile,), jnp.float32),
            pltpu.VMEM_SHARED((n_pad,), jnp.float32),
            pltpu.SemaphoreType.DMA,
        ],
    )
    def deg_kernel(dst_hbm, ew_hbm, out_hbm, idx_v, ew_v, z_v, acc, sem):
        cid = lax.axis_index("c")
        sid = lax.axis_index("s")

        def zloop(i, carry):
            z_v[pl.ds(i * LANES, LANES)] = jnp.zeros((LANES,), jnp.float32)
            return carry

        lax.fori_loop(0, rows_per_tile // LANES, zloop, 0)
        pltpu.sync_copy(z_v, acc.at[pl.ds(sid * rows_per_tile, rows_per_tile)])

        base = (cid * NSUB + sid) * k_chunks
        pltpu.sync_copy(dst_hbm.at[pl.ds(base, k_chunks)], idx_v)
        pltpu.sync_copy(ew_hbm.at[pl.ds(base, k_chunks)], ew_v)
        plsc.subcore_barrier()

        def body(g, carry):
            descs = []
            for u in range(fire):
                k = g * fire + u
                descs.append(pltpu.async_copy(
                    ew_v.at[k], acc.at[idx_v.at[k]], sem, add=True))
            for dsc in descs:
                dsc.wait()
            return carry

        lax.fori_loop(0, k_chunks // fire, body, 0)
        plsc.subcore_barrier()
        sl = pl.ds(sid * rows_per_tile, rows_per_tile)
        pltpu.sync_copy(acc.at[sl], out_hbm.at[cid, sl])

    return deg_kernel(dst2d, ew2d)


def _sc_aggregate(src, dst, ew, y, n_pad):
    d = y.shape[1]
    ep = src.shape[0]
    k_chunks = ep // (NWORK * ACHUNK)
    npairs = k_chunks // 2
    rows_per_tile = n_pad // NSUB
    half = ACHUNK // 2
    zrows = 16

    @functools.partial(
        pl.kernel,
        out_type=jax.ShapeDtypeStruct((NCORES, n_pad, d), jnp.float32),
        mesh=_sc_mesh(),
        scratch_types=[
            pltpu.VMEM((2, ACHUNK), jnp.int32),
            pltpu.VMEM((2, ACHUNK), jnp.int32),
            pltpu.VMEM((half,), jnp.int32),
            pltpu.VMEM((half,), jnp.int32),
            pltpu.VMEM((half,), jnp.int32),
            pltpu.VMEM((half,), jnp.int32),
            pltpu.VMEM((2, ACHUNK), jnp.float32),
            pltpu.VMEM((ACHUNK, 128), jnp.float32),
            pltpu.VMEM((ACHUNK, 128), jnp.float32),
            pltpu.VMEM((zrows, 128), jnp.float32),
            pltpu.VMEM_SHARED((n_pad, 128), jnp.float32),
            pltpu.SemaphoreType.DMA,
            pltpu.SemaphoreType.DMA,
            pltpu.SemaphoreType.DMA,
            pltpu.SemaphoreType.DMA,
            pltpu.SemaphoreType.DMA,
            pltpu.SemaphoreType.DMA,
            pltpu.SemaphoreType.DMA,
            pltpu.SemaphoreType.DMA,
            pltpu.SemaphoreType.DMA,
        ],
    )
    def agg_kernel(src_hbm, dst_hbm, ew_hbm, y_hbm, out_hbm,
                   sidx, didx, dp0l, dp0h, dp1l, dp1h, ew_v,
                   rows0, rows1, zbuf, acc,
                   isem0, isem1, gsem0, gsem1, s0l, s0h, s1l, s1h, zsem):
        cid = lax.axis_index("c")
        sid = lax.axis_index("s")
        rowsb = (rows0, rows1)
        dpl = (dp0l, dp1l)
        dph = (dp0h, dp1h)
        isem = (isem0, isem1)
        gsem = (gsem0, gsem1)
        ssl = (s0l, s1l)
        ssh = (s0h, s1h)

        def zrow(i, carry):
            for j in range(128 // LANES):
                zbuf[i, pl.ds(j * LANES, LANES)] = jnp.zeros((LANES,), jnp.float32)
            return carry

        lax.fori_loop(0, zrows, zrow, 0)
        zbase = sid * rows_per_tile
        for i in range(rows_per_tile // zrows):
            pltpu.async_copy(zbuf, acc.at[pl.ds(zbase + i * zrows, zrows)], zsem)

        base = (cid * NSUB + sid) * k_chunks

        def fetch_idx(k, b):
            off = (k + base) * ACHUNK
            pltpu.async_copy(src_hbm.at[pl.ds(off, ACHUNK)], sidx.at[b], isem[b])
            pltpu.async_copy(dst_hbm.at[pl.ds(off, ACHUNK)], didx.at[b], isem[b])
            pltpu.async_copy(ew_hbm.at[pl.ds(off, ACHUNK)], ew_v.at[b], isem[b])

        def wait_idx(b):
            pltpu.make_async_copy(
                src_hbm.at[pl.ds(0, ACHUNK)], sidx.at[b], isem[b]).wait()
            pltpu.make_async_copy(
                dst_hbm.at[pl.ds(0, ACHUNK)], didx.at[b], isem[b]).wait()
            pltpu.make_async_copy(
                ew_hbm.at[pl.ds(0, ACHUNK)], ew_v.at[b], isem[b]).wait()

        def scale_half(b, hf):
            rows = rowsb[b]

            def grp(g, c2):
                wv = ew_v[b, pl.ds(g * LANES, LANES)]
                for l in range(LANES):
                    w = wv[l]
                    r = g * LANES + l
                    for j in range(128 // LANES):
                        sl = pl.ds(j * LANES, LANES)
                        rows[r, sl] = rows[r, sl] * w
                return c2

            lax.fori_loop(hf * (half // LANES), (hf + 1) * (half // LANES),
                          grp, 0)

        def gather(b):
            pltpu.async_copy(y_hbm.at[sidx.at[b]], rowsb[b], gsem[b])

        def wait_gather(b):
            pltpu.make_async_copy(
                y_hbm.at[sidx.at[b]], rowsb[b], gsem[b]).wait()

        def scatter_half(b, hf):
            dp = dpl[b] if hf == 0 else dph[b]
            sem = ssl[b] if hf == 0 else ssh[b]
            for j in range(half // LANES):
                dp[pl.ds(j * LANES, LANES)] = (
                    didx[b, pl.ds(hf * half + j * LANES, LANES)])
            pltpu.async_copy(rowsb[b].at[pl.ds(hf * half, half)],
                             acc.at[dp], sem, add=True)

        def wait_scatter_half(b, hf):
            dp = dpl[b] if hf == 0 else dph[b]
            sem = ssl[b] if hf == 0 else ssh[b]
            pltpu.make_async_copy(rowsb[b].at[pl.ds(hf * half, half)],
                                  acc.at[dp], sem).wait()

        for i in range(rows_per_tile // zrows):
            pltpu.make_async_copy(
                zbuf, acc.at[pl.ds(zbase + i * zrows, zrows)], zsem).wait()
        plsc.subcore_barrier()

        fetch_idx(0, 0)
        wait_idx(0)
        gather(0)
        fetch_idx(1, 1)

        def body(j, carry):
            a = j * 2

            @pl.when(j > 0)
            def _():
                wait_scatter_half(1, 0)
                wait_scatter_half(1, 1)

            wait_idx(1)
            gather(1)
            wait_gather(0)
            scale_half(0, 0)
            scatter_half(0, 0)
            scale_half(0, 1)
            scatter_half(0, 1)

            @pl.when(j < npairs - 1)
            def _():
                fetch_idx(a + 2, 0)

            wait_gather(1)
            scale_half(1, 0)
            scatter_half(1, 0)
            wait_scatter_half(0, 0)
            wait_scatter_half(0, 1)

            @pl.when(j < npairs - 1)
            def _():
                wait_idx(0)
                gather(0)

            scale_half(1, 1)
            scatter_half(1, 1)

            @pl.when(j < npairs - 1)
            def _():
                fetch_idx(a + 3, 1)

            return carry

        lax.fori_loop(0, npairs, body, 0)
        wait_scatter_half(1, 0)
        wait_scatter_half(1, 1)
        plsc.subcore_barrier()
        sl = pl.ds(sid * rows_per_tile, rows_per_tile)
        pltpu.sync_copy(acc.at[sl], out_hbm.at[cid, sl])

    return agg_kernel(src, dst, ew, y)


def _tc_prep(xp, W, parts_t):
    n_pad, d = xp.shape
    blk = n_pad // NBLOCKS

    def body(x_ref, w_ref, p_ref, y_ref, dinv_ref):
        deg = p_ref[:, 0:1] + p_ref[:, 1:2] + 1.0
        dinv = lax.rsqrt(deg)
        xw = jnp.dot(x_ref[...], w_ref[...], preferred_element_type=jnp.float32)
        y_ref[...] = xw * dinv
        dinv_ref[...] = dinv

    return pl.pallas_call(
        body,
        grid=(NBLOCKS,),
        in_specs=[
            pl.BlockSpec((blk, d), lambda i: (i, 0)),
            pl.BlockSpec((d, d), lambda i: (0, 0)),
            pl.BlockSpec((blk, NCORES), lambda i: (i, 0)),
        ],
        out_specs=[
            pl.BlockSpec((blk, d), lambda i: (i, 0)),
            pl.BlockSpec((blk, 1), lambda i: (i, 0)),
        ],
        out_shape=[
            jax.ShapeDtypeStruct((n_pad, d), jnp.float32),
            jax.ShapeDtypeStruct((n_pad, 1), jnp.float32),
        ],
    )(xp, W, parts_t)


def _tc_post(parts, y, dinv2, b_row, n_real):
    n_pad, d = y.shape
    blk = n_pad // NBLOCKS

    def body(p_ref, y_ref, dinv_ref, b_ref, t_ref, s_ref):
        i = pl.program_id(0)
        t = dinv_ref[...] * (p_ref[0] + p_ref[1] + y_ref[...]) + b_ref[...]
        t_ref[...] = t
        rowid = lax.broadcasted_iota(jnp.int32, (blk, 1), 0) + i * blk
        tm = jnp.where(rowid < n_real, t, 0.0)

        @pl.when(i == 0)
        def _():
            s_ref[...] = jnp.zeros_like(s_ref)

        s_ref[0:1, :] += jnp.sum(tm, axis=0, keepdims=True)
        s_ref[1:2, :] += jnp.sum(tm * tm, axis=0, keepdims=True)

    return pl.pallas_call(
        body,
        grid=(NBLOCKS,),
        in_specs=[
            pl.BlockSpec((NCORES, blk, d), lambda i: (0, i, 0)),
            pl.BlockSpec((blk, d), lambda i: (i, 0)),
            pl.BlockSpec((blk, 1), lambda i: (i, 0)),
            pl.BlockSpec((1, d), lambda i: (0, 0)),
        ],
        out_specs=[
            pl.BlockSpec((blk, d), lambda i: (i, 0)),
            pl.BlockSpec((2, d), lambda i: (0, 0)),
        ],
        out_shape=[
            jax.ShapeDtypeStruct((n_pad, d), jnp.float32),
            jax.ShapeDtypeStruct((2, d), jnp.float32),
        ],
    )(parts, y, dinv2, b_row)


def _tc_bn_mm(t, stats, g_row, be_row, W, dinv2, n_real):
    n_pad, d = t.shape
    blk = n_pad // NBLOCKS
    inv_n = 1.0 / n_real

    def body(t_ref, s_ref, g_ref, be_ref, w_ref, dinv_ref, y_ref):
        m = s_ref[0:1, :] * inv_n
        var = s_ref[1:2, :] * inv_n - m * m
        scale = lax.rsqrt(var + EPS) * g_ref[...]
        h = jnp.maximum((t_ref[...] - m) * scale + be_ref[...], 0.0)
        hw = jnp.dot(h, w_ref[...], preferred_element_type=jnp.float32)
        y_ref[...] = hw * dinv_ref[...]

    return pl.pallas_call(
        body,
        grid=(NBLOCKS,),
        in_specs=[
            pl.BlockSpec((blk, d), lambda i: (i, 0)),
            pl.BlockSpec((2, d), lambda i: (0, 0)),
            pl.BlockSpec((1, d), lambda i: (0, 0)),
            pl.BlockSpec((1, d), lambda i: (0, 0)),
            pl.BlockSpec((d, d), lambda i: (0, 0)),
            pl.BlockSpec((blk, 1), lambda i: (i, 0)),
        ],
        out_specs=pl.BlockSpec((blk, d), lambda i: (i, 0)),
        out_shape=jax.ShapeDtypeStruct((n_pad, d), jnp.float32),
    )(t, stats, g_row, be_row, W, dinv2)


def _tc_head(t, stats, g_row, be_row, batch2d, Wp1, bp1_row, Wp2, bp2_row,
             n_real, n_graphs):
    n_pad, d = t.shape
    blk = n_pad // NBLOCKS
    inv_n = 1.0 / n_real
    dm = Wp1.shape[1]

    def body(t_ref, s_ref, g_ref, be_ref, b_ref, wp1_ref, bp1_ref, wp2_ref,
             bp2_ref, out_ref, s_acc, c_acc):
        i = pl.program_id(0)
        m = s_ref[0:1, :] * inv_n
        var = s_ref[1:2, :] * inv_n - m * m
        scale = lax.rsqrt(var + EPS) * g_ref[...]
        h = jnp.maximum((t_ref[...] - m) * scale + be_ref[...], 0.0)
        gids = lax.broadcasted_iota(jnp.int32, (1, n_graphs), 1)
        oh = (b_ref[...] == gids).astype(jnp.float32)
        dn = (((0,), (0,)), ((), ()))

        @pl.when(i == 0)
        def _():
            s_acc[...] = jnp.zeros_like(s_acc)
            c_acc[...] = jnp.zeros_like(c_acc)

        s_acc[...] += lax.dot_general(oh, h, dn,
                                      preferred_element_type=jnp.float32)
        c_acc[...] += lax.dot_general(oh, jnp.ones_like(h), dn,
                                      preferred_element_type=jnp.float32)

        @pl.when(i == NBLOCKS - 1)
        def _():
            pooled = s_acc[...] / jnp.maximum(c_acc[...], 1.0)
            z = jnp.dot(pooled, wp1_ref[...],
                        preferred_element_type=jnp.float32) + bp1_ref[...]
            z = jnp.maximum(z, 0.0)
            out_ref[...] = jnp.dot(z, wp2_ref[...],
                                   preferred_element_type=jnp.float32) + bp2_ref[...]

    return pl.pallas_call(
        body,
        grid=(NBLOCKS,),
        in_specs=[
            pl.BlockSpec((blk, d), lambda i: (i, 0)),
            pl.BlockSpec((2, d), lambda i: (0, 0)),
            pl.BlockSpec((1, d), lambda i: (0, 0)),
            pl.BlockSpec((1, d), lambda i: (0, 0)),
            pl.BlockSpec((blk, 1), lambda i: (i, 0)),
            pl.BlockSpec((d, dm), lambda i: (0, 0)),
            pl.BlockSpec((1, dm), lambda i: (0, 0)),
            pl.BlockSpec((dm, 1), lambda i: (0, 0)),
            pl.BlockSpec((1, 1), lambda i: (0, 0)),
        ],
        out_specs=pl.BlockSpec((n_graphs, 1), lambda i: (0, 0)),
        out_shape=jax.ShapeDtypeStruct((n_graphs, 1), jnp.float32),
        scratch_shapes=[
            pltpu.VMEM((n_graphs, d), jnp.float32),
            pltpu.VMEM((n_graphs, d), jnp.float32),
        ],
    )(t, stats, g_row, be_row, batch2d, Wp1, bp1_row, Wp2, bp2_row)


def kernel(x, edge_index, edge_attr, batch,
           W1, b1, g1, be1, W2, b2, g2, be2, Wp1, bp1, Wp2, bp2):
    n, d = x.shape
    e = edge_index.shape[1]
    n_graphs = 64

    n_pad = ((n + 2048) // 2048) * 2048
    estep = NWORK * DCHUNK * 8
    ep = ((e + estep - 1) // estep) * estep

    npad_rows = n_pad - n
    pad_src = jnp.arange(ep - e, dtype=jnp.int32)
    pad_dst = n + jnp.broadcast_to(
        jnp.arange(npad_rows, dtype=jnp.int32),
        ((ep - e + npad_rows - 1) // npad_rows, npad_rows)).reshape(-1)[:ep - e]
    src = jnp.concatenate([edge_index[0], pad_src])
    dst = jnp.concatenate([edge_index[1], pad_dst])
    ew = jnp.concatenate(
        [edge_attr[:, 0], jnp.zeros((ep - e,), jnp.float32)])

    dst2d = dst.reshape(ep // DCHUNK, DCHUNK)
    ew2d = ew.reshape(ep // DCHUNK, DCHUNK)

    xp = jnp.pad(x, ((0, n_pad - n), (0, 0)))
    batch2d = jnp.pad(batch, (0, n_pad - n),
                      constant_values=n_graphs).reshape(n_pad, 1)

    b1r, g1r, be1r = b1.reshape(1, d), g1.reshape(1, d), be1.reshape(1, d)
    b2r, g2r, be2r = b2.reshape(1, d), g2.reshape(1, d), be2.reshape(1, d)
    bp1r = bp1.reshape(1, -1)
    bp2r = bp2.reshape(1, 1)

    deg_parts = _sc_degree(dst2d, ew2d, n_pad)
    parts_t = deg_parts.T

    y1, dinv2 = _tc_prep(xp, W1, parts_t)
    p1 = _sc_aggregate(src, dst, ew, y1, n_pad)
    t1, s1 = _tc_post(p1, y1, dinv2, b1r, n)
    y2 = _tc_bn_mm(t1, s1, g1r, be1r, W2, dinv2, n)
    p2 = _sc_aggregate(src, dst, ew, y2, n_pad)
    t2, s2 = _tc_post(p2, y2, dinv2, b2r, n)
    out = _tc_head(t2, s2, g2r, be2r, batch2d, Wp1, bp1r, Wp2, bp2r,
                   n, n_graphs)
    return out
